# baseline (speedup 1.0000x reference)
import functools

import jax
import jax.numpy as jnp
from jax import lax
from jax.experimental import pallas as pl
from jax.experimental.pallas import tpu as pltpu
from jax.experimental.pallas import tpu_sc as plsc

NC = 2
NS = 16
NW = NC * NS
LANES = 16
SCHUNK = 8
HCHUNK = 2048
NSLOT = 3
SC_B = 3


def _make_sc_add(B, S, H):
    pos_per_w = S // NW
    n_sc = pos_per_w // SCHUNK
    n_hc = H // HCHUNK
    n_tiles = n_sc * n_hc
    n_vec = HCHUNK // LANES

    mesh = plsc.VectorSubcoreMesh(core_axis_name="c", subcore_axis_name="s")

    @functools.partial(
        pl.kernel,
        mesh=mesh,
        out_type=jax.ShapeDtypeStruct((1, S, H), jnp.float32),
        scratch_types=(
            [pltpu.VMEM((SCHUNK, HCHUNK), jnp.float32)] * NSLOT
            + [pltpu.VMEM((SCHUNK, HCHUNK), jnp.float32)] * NSLOT
            + [pltpu.SemaphoreType.DMA] * (3 * NSLOT)
        ),
        compiler_params=pltpu.CompilerParams(use_tc_tiling_on_sc=True),
    )
    def sc_add(x_hbm, p_hbm, o_hbm, *bufs):
        pos_v = bufs[0:NSLOT]
        buf_v = bufs[NSLOT:2 * NSLOT]
        psem = bufs[2 * NSLOT:3 * NSLOT]
        isem = bufs[3 * NSLOT:4 * NSLOT]
        osem = bufs[4 * NSLOT:5 * NSLOT]
        wid = lax.axis_index("s") * NC + lax.axis_index("c")
        s_base = wid * pos_per_w

        def tile_slices(t):
            c, hi = divmod(t, n_hc)
            s0 = s_base + c * SCHUNK
            return pl.ds(s0, SCHUNK), pl.ds(hi * HCHUNK, HCHUNK)

        def start_in(t):
            sl = t % NSLOT
            ssl, hsl = tile_slices(t)
            return [
                pltpu.async_copy(p_hbm.at[ssl, hsl], pos_v[sl], psem[sl]),
                pltpu.async_copy(x_hbm.at[SC_B, ssl, hsl], buf_v[sl], isem[sl]),
            ]

        def start_out(t):
            sl = t % NSLOT
            ssl, hsl = tile_slices(t)
            return [pltpu.async_copy(
                buf_v[sl], o_hbm.at[0, ssl, hsl], osem[sl])]

        def compute(sl):
            @plsc.parallel_loop(0, n_vec)
            def body(j):
                sli = pl.ds(j * LANES, LANES)
                for s in range(SCHUNK):
                    buf_v[sl][s, sli] = buf_v[sl][s, sli] + pos_v[sl][s, sli]

        ins = {0: start_in(0), 1: start_in(1)}
        outs = {}
        for t in range(n_tiles):
            for h in ins.pop(t):
                h.wait()
            compute(t % NSLOT)
            outs[t] = start_out(t)
            if t + 2 < n_tiles:
                if t >= 1:
                    for h in outs.pop(t - 1):
                        h.wait()
                ins[t + 2] = start_in(t + 2)
        for t in sorted(outs):
            for h in outs.pop(t):
                h.wait()

    return sc_add


def _tc_add(concat_embeddings, pos_table, nb):
    B, S, H = concat_embeddings.shape
    BS = 256

    def body(x_ref, p_ref, o_ref):
        o_ref[...] = x_ref[...] + p_ref[...]

    return pl.pallas_call(
        body,
        grid=(S // BS, nb),
        in_specs=[
            pl.BlockSpec((1, BS, H), lambda j, b: (b, j, 0)),
            pl.BlockSpec((BS, H), lambda j, b: (j, 0)),
        ],
        out_specs=pl.BlockSpec((1, BS, H), lambda j, b: (b, j, 0)),
        out_shape=jax.ShapeDtypeStruct((nb, S, H), concat_embeddings.dtype),
    )(concat_embeddings, pos_table)


def kernel(concat_embeddings, pos_table):
    B, S, H = concat_embeddings.shape
    sc_add = _make_sc_add(B, S, H)
    sc_part = sc_add(concat_embeddings, pos_table)
    tc_part = _tc_add(concat_embeddings, pos_table, B - 1)
    return jnp.concatenate([tc_part, sc_part], axis=0)


# speedup vs baseline: 1.5714x; 1.5714x over previous
import functools

import jax
import jax.numpy as jnp
from jax import lax
from jax.experimental import pallas as pl
from jax.experimental.pallas import tpu as pltpu
from jax.experimental.pallas import tpu_sc as plsc

NC = 2
NS = 16
NW = NC * NS
LANES = 16
SCHUNK = 8
HCHUNK = 512
NSLOT = 5


def _make_sc_add(B, S, H):
    pos_per_w = S // NW
    n_sc = pos_per_w // SCHUNK
    n_hc = H // HCHUNK
    n_tiles = n_sc * n_hc
    n_vec = HCHUNK // LANES

    mesh = plsc.VectorSubcoreMesh(core_axis_name="c", subcore_axis_name="s")

    @functools.partial(
        pl.kernel,
        mesh=mesh,
        out_type=jax.ShapeDtypeStruct((B, S, H), jnp.float32),
        scratch_types=(
            [pltpu.VMEM((SCHUNK, HCHUNK), jnp.float32)] * NSLOT
            + [pltpu.VMEM((B, SCHUNK, HCHUNK), jnp.float32)] * NSLOT
            + [pltpu.SemaphoreType.DMA] * (3 * NSLOT)
        ),
        compiler_params=pltpu.CompilerParams(use_tc_tiling_on_sc=True),
    )
    def sc_add(x_hbm, p_hbm, o_hbm, *bufs):
        pos_v = bufs[0:NSLOT]
        buf_v = bufs[NSLOT:2 * NSLOT]
        psem = bufs[2 * NSLOT:3 * NSLOT]
        isem = bufs[3 * NSLOT:4 * NSLOT]
        osem = bufs[4 * NSLOT:5 * NSLOT]
        wid = lax.axis_index("s") * NC + lax.axis_index("c")
        s_base = wid * pos_per_w

        def tile_slices(t):
            c, hi = divmod(t, n_hc)
            s0 = s_base + c * SCHUNK
            return pl.ds(s0, SCHUNK), pl.ds(hi * HCHUNK, HCHUNK)

        def start_in(t):
            sl = t % NSLOT
            ssl, hsl = tile_slices(t)
            return [
                pltpu.async_copy(p_hbm.at[ssl, hsl], pos_v[sl], psem[sl]),
                pltpu.async_copy(x_hbm.at[:, ssl, hsl], buf_v[sl], isem[sl]),
            ]

        def start_out(t):
            sl = t % NSLOT
            ssl, hsl = tile_slices(t)
            return [pltpu.async_copy(
                buf_v[sl], o_hbm.at[:, ssl, hsl], osem[sl])]

        def compute(sl):
            @plsc.parallel_loop(0, n_vec * SCHUNK)
            def body(i):
                s = i & (SCHUNK - 1)
                j = i >> 3
                sli = pl.ds(j * LANES, LANES)
                pv = pos_v[sl][s, sli]
                for b in range(B):
                    buf_v[sl][b, s, sli] = buf_v[sl][b, s, sli] + pv

        ins = {0: start_in(0), 1: start_in(1)}
        outs = {}
        for t in range(n_tiles):
            for h in ins.pop(t):
                h.wait()
            compute(t % NSLOT)
            outs[t] = start_out(t)
            if t + 2 < n_tiles:
                if t >= 1:
                    for h in outs.pop(t - 1):
                        h.wait()
                ins[t + 2] = start_in(t + 2)
        for t in sorted(outs):
            for h in outs.pop(t):
                h.wait()

    return sc_add


def kernel(concat_embeddings, pos_table):
    B, S, H = concat_embeddings.shape
    sc_add = _make_sc_add(B, S, H)
    return sc_add(concat_embeddings, pos_table)
